# Initial kernel scaffold; baseline (speedup 1.0000x reference)
import jax
import jax.numpy as jnp
from jax.experimental import pallas as pl


def kernel(seq_inputs, seq_lens, seq_mask, edge_index, edge_type, cand_start, cand_end, emb_table, pos_table, W_f, U_f, b_f, W_b, U_b, b_b, rel_embed, W_src1, W_dst1, W_rel1, a1, W_res1, W_src2, W_dst2, a2, ln_g, ln_b, W_out, b_out):
    raise NotImplementedError("write your pallas kernel here")


# speedup vs baseline: 16.9229x; 16.9229x over previous
import functools

import jax
import jax.numpy as jnp
from jax import lax
from jax.experimental import pallas as pl
from jax.experimental.pallas import tpu as pltpu
from jax.experimental.pallas import tpu_sc as plsc

B, L = 20, 500
D = 128
H, DH = 8, 16
E = 320000
N = B * L
NP = 10240
RSPAN = 2560
BLK = 512
ECAP = E + 4 * BLK
NCAND = 2048
HOP = 4
ALPHA = 0.15
A1M = 1.0 - ALPHA
BPAD = 24
HIGH = lax.Precision.HIGHEST

_SC_PARAMS = pltpu.CompilerParams(needs_layout_passes=False,
                                  use_tc_tiling_on_sc=False)
_MESH = plsc.VectorSubcoreMesh(core_axis_name="c", subcore_axis_name="s")


@functools.partial(
    pl.kernel, mesh=_MESH, compiler_params=_SC_PARAMS,
    out_type=jax.ShapeDtypeStruct((12288, D), jnp.float32),
    scratch_types=[
        pltpu.VMEM((384,), jnp.int32),
        pltpu.VMEM((384, D), jnp.float32),
        pltpu.SemaphoreType.DMA,
    ],
)
def _emb_kernel(emb_hbm, idx_hbm, out_hbm, idx_v, rows_v, sem):
    wid = lax.axis_index("s") * 2 + lax.axis_index("c")
    base = pl.multiple_of(wid * 384, 8)
    pltpu.sync_copy(idx_hbm.at[pl.ds(base, 384)], idx_v)
    cps = [pltpu.async_copy(emb_hbm.at[idx_v.at[pl.ds(k * 128, 128)]],
                            rows_v.at[pl.ds(k * 128, 128), :], sem)
           for k in range(3)]
    for c in cps:
        c.wait()
    pltpu.sync_copy(rows_v, out_hbm.at[pl.ds(base, 384), :])


def _lstm_body(xg, pos, wuf, bf, wub, bb, out, hf, cf, hb, cb):
    z24 = jnp.zeros((BPAD, D), jnp.float32)
    hf[...] = z24
    cf[...] = z24
    hb[...] = z24
    cb[...] = z24

    def gates(cat, wu, b):
        z = jnp.dot(cat, wu[...], preferred_element_type=jnp.float32,
                    precision=HIGH) + b[...][None, :]
        zi = z[:, 0:D]
        zf = z[:, D:2 * D]
        zg = z[:, 2 * D:3 * D]
        zo = z[:, 3 * D:4 * D]
        i = jax.nn.sigmoid(zi)
        f = jax.nn.sigmoid(zf)
        g = jnp.tanh(zg)
        o = jax.nn.sigmoid(zo)
        return i, f, g, o

    def step(t, _):
        tb = (L - 1) - t
        xt = xg[t] + pos[t]
        xb = xg[tb] + pos[tb]
        catf = jnp.concatenate([xt, hf[...]], axis=1)
        catb = jnp.concatenate([xb, hb[...]], axis=1)
        fi, ff, fg, fo = gates(catf, wuf, bf)
        bi, bf_, bg, bo = gates(catb, wub, bb)
        cfn = ff * cf[...] + fi * fg
        hfn = fo * jnp.tanh(cfn)
        cbn = bf_ * cb[...] + bi * bg
        hbn = bo * jnp.tanh(cbn)
        cf[...] = cfn
        hf[...] = hfn
        cb[...] = cbn
        hb[...] = hbn
        out[t, :, 0:D] = hfn
        out[tb, :, D:2 * D] = hbn
        return 0

    lax.fori_loop(0, L, step, 0)


def _lstm(xg, pos, wuf, bf, wub, bb):
    return pl.pallas_call(
        _lstm_body,
        out_shape=jax.ShapeDtypeStruct((L, BPAD, 2 * D), jnp.float32),
        scratch_shapes=[pltpu.VMEM((BPAD, D), jnp.float32) for _ in range(4)],
    )(xg, pos, wuf, bf, wub, bb)


RB = NP // 4


def _proj1_body(node, wcat, rel, wrelf, y, r1):
    y[...] = jnp.dot(node[...], wcat[...],
                     preferred_element_type=jnp.float32, precision=HIGH)

    @pl.when(pl.program_id(0) == 0)
    def _():
        r1[...] = jnp.dot(rel[...], wrelf[...],
                          preferred_element_type=jnp.float32, precision=HIGH)


def _proj1(node, wcat, rel, wrelf):
    return pl.pallas_call(
        _proj1_body,
        grid=(4,),
        in_specs=[
            pl.BlockSpec((RB, 2 * D), lambda i: (i, 0)),
            pl.BlockSpec((2 * D, 272), lambda i: (0, 0)),
            pl.BlockSpec((16, 64), lambda i: (0, 0)),
            pl.BlockSpec((64, 8), lambda i: (0, 0)),
        ],
        out_specs=(pl.BlockSpec((RB, 272), lambda i: (i, 0)),
                   pl.BlockSpec((16, 8), lambda i: (0, 0))),
        out_shape=(jax.ShapeDtypeStruct((NP, 272), jnp.float32),
                   jax.ShapeDtypeStruct((16, 8), jnp.float32)),
    )(node, wcat, rel, wrelf)


def _mid_body(out1, res1, wcat, x2, y2):
    v = out1[...] + res1[...]
    x = jnp.where(v > 0, v, jnp.exp(v) - 1.0)
    x2[...] = x
    y2[...] = jnp.dot(x, wcat[...], preferred_element_type=jnp.float32,
                      precision=HIGH)


def _mid(out1, res1, wcat):
    return pl.pallas_call(
        _mid_body,
        grid=(4,),
        in_specs=[
            pl.BlockSpec((RB, D), lambda i: (i, 0)),
            pl.BlockSpec((RB, D), lambda i: (i, 0)),
            pl.BlockSpec((D, 144), lambda i: (0, 0)),
        ],
        out_specs=(pl.BlockSpec((RB, D), lambda i: (i, 0)),
                   pl.BlockSpec((RB, 144), lambda i: (i, 0))),
        out_shape=(jax.ShapeDtypeStruct((NP, D), jnp.float32),
                   jax.ShapeDtypeStruct((NP, 144), jnp.float32)),
    )(out1, res1, wcat)


def _fin_body(out2, x2, lng, lnb, wout, y):
    v = out2[...] + x2[...]
    x = jnp.where(v > 0, v, jnp.exp(v) - 1.0)
    mu = jnp.mean(x, axis=1, keepdims=True)
    xc = x - mu
    var = jnp.mean(xc * xc, axis=1, keepdims=True)
    xn = xc * lax.rsqrt(var + 1e-5) * lng[...][None, :] + lnb[...][None, :]
    y[...] = jnp.dot(xn, wout[...], preferred_element_type=jnp.float32,
                     precision=HIGH)


def _fin(out2, x2, lng, lnb, wout8):
    return pl.pallas_call(
        _fin_body,
        grid=(4,),
        in_specs=[
            pl.BlockSpec((RB, D), lambda i: (i, 0)),
            pl.BlockSpec((RB, D), lambda i: (i, 0)),
            pl.BlockSpec((D,), lambda i: (0,)),
            pl.BlockSpec((D,), lambda i: (0,)),
            pl.BlockSpec((D, 8), lambda i: (0, 0)),
        ],
        out_specs=pl.BlockSpec((RB, 8), lambda i: (i, 0)),
        out_shape=jax.ShapeDtypeStruct((NP, 8), jnp.float32),
    )(out2, x2, lng, lnb, wout8)


def _gdt_layer_kernel(has_rel):
    nsc = 10 if has_rel else 8

    def body(*refs):
        if has_rel:
            (s_tab, t_tab, rel_hm, src_h, dst_h, et_h, offs_h, f0_h,
             attn_h, fa_h, fo_h,
             s_loc, t_loc, r_loc, den, f0_loc, out2d, rows2d,
             src_v, dst_v, et_v, ex_v, attn_v, idx_v, offs_v, sem) = refs
        else:
            (s_tab, t_tab, src_h, dst_h, offs_h, f0_h,
             attn_h, fa_h, fo_h,
             s_loc, t_loc, den, f0_loc, out2d, rows2d,
             src_v, dst_v, ex_v, attn_v, idx_v, offs_v, sem) = refs
            rel_hm = et_h = r_loc = et_v = None

        cid = lax.axis_index("c")
        sid = lax.axis_index("s")
        h = cid * 4 + lax.rem(sid, 4)
        r = lax.div(sid, 4)
        nbase = r * RSPAN

        pltpu.sync_copy(s_tab.at[h], s_loc)
        pltpu.sync_copy(t_tab.at[h], t_loc)
        if has_rel:
            pltpu.sync_copy(rel_hm.at[h], r_loc)
        pltpu.sync_copy(offs_h, offs_v)
        rowbase = pl.multiple_of(h * NP + nbase, 8)
        pltpu.sync_copy(f0_h.at[pl.ds(rowbase, RSPAN), :], f0_loc)

        lanes = lax.iota(jnp.int32, 16)
        ov = offs_v[...]
        start = jnp.sum(jnp.where(lanes == r, ov, 0))
        count = jnp.sum(jnp.where(lanes == r + 4, ov, 0))
        n16 = jnp.sum(jnp.where(lanes == 8, ov, 0))
        nrs = jnp.sum(jnp.where(lanes == 9, ov, 0))
        ng = jnp.sum(jnp.where(lanes == 10, ov, 0))
        nhop = jnp.sum(jnp.where(lanes == 11, ov, 0))
        nblk = lax.shift_right_logical(count + (BLK - 1), 9)

        def zden(i, _):
            den[pl.ds(i * 16, 16)] = jnp.zeros((16,), jnp.float32)
            return 0
        lax.fori_loop(0, n16, zden, 0)

        def p1(b, _):
            ebase = pl.multiple_of(start + b * BLK, 8)
            pltpu.sync_copy(src_h.at[pl.ds(ebase, BLK)], src_v)
            pltpu.sync_copy(dst_h.at[pl.ds(ebase, BLK)], dst_v)
            if has_rel:
                pltpu.sync_copy(et_h.at[pl.ds(ebase, BLK)], et_v)
            def p1g(i, _):
                sl = pl.ds(pl.multiple_of(i * 16, 16), 16)
                sv = src_v[sl]
                dv = dst_v[sl]
                z = plsc.load_gather(s_loc, [sv]) + plsc.load_gather(t_loc, [dv])
                if has_rel:
                    z = z + plsc.load_gather(r_loc, [et_v[sl]])
                zl = jnp.where(z >= 0, z, 0.2 * z)
                ex = jnp.exp(zl)
                ex_v[sl] = ex
                msk = (b * BLK + i * 16 + lanes) < count
                plsc.addupdate_scatter(den, [dv - nbase], ex, mask=msk)
                return 0
            lax.fori_loop(0, ng, p1g, 0)
            pltpu.sync_copy(ex_v, attn_h.at[h, pl.ds(ebase, BLK)])
            return 0
        lax.fori_loop(0, nblk, p1, 0)

        def dinv(i, _):
            sl = pl.ds(i * 16, 16)
            den[sl] = 1.0 / (den[sl] + 1e-16)
            return 0
        lax.fori_loop(0, n16, dinv, 0)

        def p2(b, _):
            ebase = pl.multiple_of(start + b * BLK, 8)
            pltpu.sync_copy(dst_h.at[pl.ds(ebase, BLK)], dst_v)
            pltpu.sync_copy(attn_h.at[h, pl.ds(ebase, BLK)], ex_v)
            def p2g(i, _):
                sl = pl.ds(pl.multiple_of(i * 16, 16), 16)
                dl = dst_v[sl] - nbase
                msk = (b * BLK + i * 16 + lanes) < count
                dv = plsc.load_gather(den, [dl], mask=msk)
                attn_v[sl] = ex_v[sl] * dv
                return 0
            lax.fori_loop(0, ng, p2g, 0)
            pltpu.sync_copy(attn_v, attn_h.at[h, pl.ds(ebase, BLK)])
            return 0
        lax.fori_loop(0, nblk, p2, 0)

        pltpu.sync_copy(f0_loc, fa_h.at[pl.ds(rowbase, RSPAN), :])
        plsc.subcore_barrier()

        def hoploop(k, _):
            def zout(i, _):
                out2d[i] = jnp.zeros((16,), jnp.float32)
                return 0
            lax.fori_loop(0, nrs, zout, 0)

            def hop(b, _):
                ebase = pl.multiple_of(start + b * BLK, 8)
                pltpu.sync_copy(src_h.at[pl.ds(ebase, BLK)], src_v)
                pltpu.sync_copy(dst_h.at[pl.ds(ebase, BLK)], dst_v)
                pltpu.sync_copy(attn_h.at[h, pl.ds(ebase, BLK)], attn_v)
                hoff = h * NP
                def idxg(i, _):
                    sl = pl.ds(pl.multiple_of(i * 16, 16), 16)
                    idx_v[sl] = src_v[sl] + hoff
                    return 0
                lax.fori_loop(0, ng, idxg, 0)
                cps = [pltpu.async_copy(
                    fa_h.at[idx_v.at[pl.ds(kk * 128, 128)]],
                    rows2d.at[pl.ds(kk * 128, 128), :], sem)
                    for kk in range(BLK // 128)]
                for c in cps:
                    c.wait()
                def hopg(i, _):
                    sl = pl.ds(pl.multiple_of(i * 16, 16), 16)
                    dl = dst_v[sl] - nbase
                    av = attn_v[sl]
                    msk = (b * BLK + i * 16 + lanes) < count
                    rbase = i * 16
                    for j in range(16):
                        fv = plsc.load_gather(
                            rows2d, [rbase + lanes,
                                     jnp.full((16,), j, jnp.int32)])
                        plsc.addupdate_scatter(
                            out2d, [dl, jnp.full((16,), j, jnp.int32)],
                            fv * av, mask=msk)
                    return 0
                lax.fori_loop(0, ng, hopg, 0)
                return 0
            lax.fori_loop(0, nblk, hop, 0)

            def comb(i, _):
                out2d[i] = ALPHA * f0_loc[i] + A1M * out2d[i]
                return 0
            lax.fori_loop(0, nrs, comb, 0)
            plsc.subcore_barrier()
            pltpu.sync_copy(out2d, fa_h.at[pl.ds(rowbase, RSPAN), :])
            plsc.subcore_barrier()
            return 0

        lax.fori_loop(0, nhop, hoploop, 0)
        pltpu.sync_copy(out2d, fo_h.at[pl.ds(rowbase, RSPAN), :])

    scr = [
        pltpu.VMEM((NP,), jnp.float32),
        pltpu.VMEM((NP,), jnp.float32),
    ]
    if has_rel:
        scr.append(pltpu.VMEM((16,), jnp.float32))
    scr += [
        pltpu.VMEM((RSPAN,), jnp.float32),
        pltpu.VMEM((RSPAN, DH), jnp.float32),
        pltpu.VMEM((RSPAN, DH), jnp.float32),
        pltpu.VMEM((BLK, DH), jnp.float32),
        pltpu.VMEM((BLK,), jnp.int32),
        pltpu.VMEM((BLK,), jnp.int32),
    ]
    if has_rel:
        scr.append(pltpu.VMEM((BLK,), jnp.int32))
    scr += [
        pltpu.VMEM((BLK,), jnp.float32),
        pltpu.VMEM((BLK,), jnp.float32),
        pltpu.VMEM((BLK,), jnp.int32),
        pltpu.VMEM((16,), jnp.int32),
        pltpu.SemaphoreType.DMA,
    ]
    out_type = (
        jax.ShapeDtypeStruct((H, ECAP), jnp.float32),
        jax.ShapeDtypeStruct((H * NP, DH), jnp.float32),
        jax.ShapeDtypeStruct((H * NP, DH), jnp.float32),
    )
    return pl.kernel(body, mesh=_MESH, compiler_params=_SC_PARAMS,
                     out_type=out_type, scratch_types=scr)


_gdt_l1 = _gdt_layer_kernel(True)
_gdt_l2 = _gdt_layer_kernel(False)


@functools.partial(
    pl.kernel, mesh=_MESH, compiler_params=_SC_PARAMS,
    out_type=jax.ShapeDtypeStruct((NCAND,), jnp.float32),
    scratch_types=[
        pltpu.VMEM((NP,), jnp.float32),
        pltpu.VMEM((64,), jnp.int32),
        pltpu.VMEM((64,), jnp.int32),
        pltpu.VMEM((64,), jnp.float32),
        pltpu.VMEM((16,), jnp.float32),
    ],
)
def _cand_kernel(nw_hbm, cs_hbm, ce_hbm, bout_hbm, out_hbm,
                 nw_loc, cs_v, ce_v, o_v, b_v):
    wid = lax.axis_index("s") * 2 + lax.axis_index("c")
    base = pl.multiple_of(wid * 64, 8)
    pltpu.sync_copy(nw_hbm, nw_loc)
    pltpu.sync_copy(cs_hbm.at[pl.ds(base, 64)], cs_v)
    pltpu.sync_copy(ce_hbm.at[pl.ds(base, 64)], ce_v)
    pltpu.sync_copy(bout_hbm, b_v)
    bv = b_v[...]
    for i in range(4):
        sl = pl.ds(i * 16, 16)
        sv = plsc.load_gather(nw_loc, [cs_v[sl] + 1])
        ev = plsc.load_gather(nw_loc, [ce_v[sl] - 1])
        o_v[sl] = 0.5 * (sv + ev) + bv
    pltpu.sync_copy(o_v, out_hbm.at[pl.ds(base, 64)])


def _fold(w, a):
    return (w.reshape(-1, H, DH) * a[None]).sum(-1)


def kernel(seq_inputs, seq_lens, seq_mask, edge_index, edge_type, cand_start,
           cand_end, emb_table, pos_table, W_f, U_f, b_f, W_b, U_b, b_b,
           rel_embed, W_src1, W_dst1, W_rel1, a1, W_res1, W_src2, W_dst2, a2,
           ln_g, ln_b, W_out, b_out):
    f32 = jnp.float32
    src = edge_index[0].astype(jnp.int32)
    dst = edge_index[1].astype(jnp.int32)
    et = edge_type.astype(jnp.int32)

    r_e = dst // RSPAN
    onehot = (r_e[:, None] == jnp.arange(4)[None, :]).astype(jnp.int32)
    counts = onehot.sum(0)
    rank = jnp.take_along_axis(jnp.cumsum(onehot, axis=0),
                               r_e[:, None], axis=1)[:, 0] - 1
    cap = ((counts + BLK - 1) // BLK) * BLK
    offs_p = jnp.concatenate([jnp.zeros((1,), jnp.int32),
                              jnp.cumsum(cap)]).astype(jnp.int32)
    pos_e = offs_p[r_e] + rank
    slot = jnp.arange(ECAP, dtype=jnp.int32)
    r_slot = (slot[:, None] >= offs_p[None, 1:4]).astype(jnp.int32).sum(1)
    packed = jnp.stack([src, dst, et], axis=1)
    dflt = jnp.stack([jnp.zeros((ECAP,), jnp.int32), r_slot * RSPAN,
                      jnp.zeros((ECAP,), jnp.int32)], axis=1)
    packed_s = dflt.at[pos_e].set(packed)
    src_s, dst_s, et_s = packed_s[:, 0], packed_s[:, 1], packed_s[:, 2]
    offs16 = jnp.zeros((16,), jnp.int32).at[:4].set(offs_p[:4]).at[4:8].set(counts).at[8:12].set(
        jnp.array([RSPAN // 16, RSPAN, BLK // 16, HOP], jnp.int32))

    idx_tm = jnp.pad(seq_inputs.astype(jnp.int32).T, ((0, 0), (0, BPAD - B)))
    idx_tm = jnp.pad(idx_tm.reshape(-1), (0, 288))
    xg = _emb_kernel(emb_table.astype(f32), idx_tm)[:L * BPAD].reshape(
        L, BPAD, D)
    wuf = jnp.concatenate([W_f, U_f], axis=0).astype(f32)
    wub = jnp.concatenate([W_b, U_b], axis=0).astype(f32)
    hcat = _lstm(xg, pos_table.astype(f32), wuf, b_f.astype(f32), wub,
                 b_b.astype(f32))
    node = hcat.transpose(1, 0, 2)[:B].reshape(N, 2 * D)
    node = jnp.pad(node, ((0, NP - N), (0, 0)))

    ws1 = _fold(W_src1, a1)
    wt1 = _fold(W_dst1, a1)
    wrelf = _fold(W_rel1, a1)
    wcat1 = jnp.concatenate([ws1, wt1, W_dst1, W_res1], axis=1).astype(f32)
    y1, r1 = _proj1(node, wcat1, rel_embed.astype(f32), wrelf.astype(f32))
    s1t = y1[:, 0:8].T
    t1t = y1[:, 8:16].T
    f01 = y1[:, 16:144].reshape(NP, H, DH).transpose(1, 0, 2).reshape(H * NP, DH)
    res1 = y1[:, 144:272]
    rel_hm = r1.T

    _, _, feat1 = _gdt_l1(s1t, t1t, rel_hm, src_s, dst_s, et_s, offs16, f01)
    out1 = feat1.reshape(H, NP, DH).transpose(1, 0, 2).reshape(NP, H * DH)

    ws2 = _fold(W_src2, a2)
    wt2 = _fold(W_dst2, a2)
    wcat2 = jnp.concatenate([ws2, wt2, W_dst2], axis=1).astype(f32)
    x2, y2 = _mid(out1, res1, wcat2)
    s2t = y2[:, 0:8].T
    t2t = y2[:, 8:16].T
    f02 = y2[:, 16:144].reshape(NP, H, DH).transpose(1, 0, 2).reshape(H * NP, DH)

    _, _, feat2 = _gdt_l2(s2t, t2t, src_s, dst_s, offs16, f02)
    out2 = feat2.reshape(H, NP, DH).transpose(1, 0, 2).reshape(NP, H * DH)

    wout8 = jnp.pad(W_out.astype(f32), ((0, 0), (0, 7)))
    y3 = _fin(out2, x2, ln_g.astype(f32), ln_b.astype(f32), wout8)
    nodew = y3[:, 0]
    bvec = jnp.broadcast_to(b_out.astype(f32), (16,))
    cand = _cand_kernel(nodew, cand_start.astype(jnp.int32),
                        cand_end.astype(jnp.int32), bvec)
    return cand[:, None]


# speedup vs baseline: 19.2820x; 1.1394x over previous
import functools

import jax
import jax.numpy as jnp
from jax import lax
from jax.experimental import pallas as pl
from jax.experimental.pallas import tpu as pltpu
from jax.experimental.pallas import tpu_sc as plsc

B, L = 20, 500
D = 128
H, DH = 8, 16
E = 320000
N = B * L
NP = 10240
RSPAN = 2560
BLK = 2048
ECAP = E + 4 * BLK
NCAND = 2048
HOP = 4
ALPHA = 0.15
A1M = 1.0 - ALPHA
BPAD = 24
HIGH = lax.Precision.HIGHEST

_SC_PARAMS = pltpu.CompilerParams(needs_layout_passes=False,
                                  use_tc_tiling_on_sc=False)
_MESH = plsc.VectorSubcoreMesh(core_axis_name="c", subcore_axis_name="s")


@functools.partial(
    pl.kernel, mesh=_MESH, compiler_params=_SC_PARAMS,
    out_type=jax.ShapeDtypeStruct((12288, D), jnp.float32),
    scratch_types=[
        pltpu.VMEM((384,), jnp.int32),
        pltpu.VMEM((384, D), jnp.float32),
        pltpu.SemaphoreType.DMA,
    ],
)
def _emb_kernel(emb_hbm, idx_hbm, out_hbm, idx_v, rows_v, sem):
    wid = lax.axis_index("s") * 2 + lax.axis_index("c")
    base = pl.multiple_of(wid * 384, 8)
    pltpu.sync_copy(idx_hbm.at[pl.ds(base, 384)], idx_v)
    cps = [pltpu.async_copy(emb_hbm.at[idx_v.at[pl.ds(k * 128, 128)]],
                            rows_v.at[pl.ds(k * 128, 128), :], sem)
           for k in range(3)]
    for c in cps:
        c.wait()
    pltpu.sync_copy(rows_v, out_hbm.at[pl.ds(base, 384), :])


def _lstm_body(xg, pos, wuf, bf, wub, bb, out, hf, cf, hb, cb):
    z24 = jnp.zeros((BPAD, D), jnp.float32)
    hf[...] = z24
    cf[...] = z24
    hb[...] = z24
    cb[...] = z24

    def gates(cat, wu, b):
        z = jnp.dot(cat, wu[...], preferred_element_type=jnp.float32,
                    precision=HIGH) + b[...][None, :]
        zi = z[:, 0:D]
        zf = z[:, D:2 * D]
        zg = z[:, 2 * D:3 * D]
        zo = z[:, 3 * D:4 * D]
        i = jax.nn.sigmoid(zi)
        f = jax.nn.sigmoid(zf)
        g = jnp.tanh(zg)
        o = jax.nn.sigmoid(zo)
        return i, f, g, o

    def step(t, _):
        tb = (L - 1) - t
        xt = xg[t] + pos[t]
        xb = xg[tb] + pos[tb]
        catf = jnp.concatenate([xt, hf[...]], axis=1)
        catb = jnp.concatenate([xb, hb[...]], axis=1)
        fi, ff, fg, fo = gates(catf, wuf, bf)
        bi, bf_, bg, bo = gates(catb, wub, bb)
        cfn = ff * cf[...] + fi * fg
        hfn = fo * jnp.tanh(cfn)
        cbn = bf_ * cb[...] + bi * bg
        hbn = bo * jnp.tanh(cbn)
        cf[...] = cfn
        hf[...] = hfn
        cb[...] = cbn
        hb[...] = hbn
        out[t, :, 0:D] = hfn
        out[tb, :, D:2 * D] = hbn
        return 0

    lax.fori_loop(0, L, step, 0)


def _lstm(xg, pos, wuf, bf, wub, bb):
    return pl.pallas_call(
        _lstm_body,
        out_shape=jax.ShapeDtypeStruct((L, BPAD, 2 * D), jnp.float32),
        scratch_shapes=[pltpu.VMEM((BPAD, D), jnp.float32) for _ in range(4)],
    )(xg, pos, wuf, bf, wub, bb)


RB = NP // 4


def _proj1_body(node, wcat, rel, wrelf, y, r1):
    y[...] = jnp.dot(node[...], wcat[...],
                     preferred_element_type=jnp.float32, precision=HIGH)

    @pl.when(pl.program_id(0) == 0)
    def _():
        r1[...] = jnp.dot(rel[...], wrelf[...],
                          preferred_element_type=jnp.float32, precision=HIGH)


def _proj1(node, wcat, rel, wrelf):
    return pl.pallas_call(
        _proj1_body,
        grid=(4,),
        in_specs=[
            pl.BlockSpec((RB, 2 * D), lambda i: (i, 0)),
            pl.BlockSpec((2 * D, 272), lambda i: (0, 0)),
            pl.BlockSpec((16, 64), lambda i: (0, 0)),
            pl.BlockSpec((64, 8), lambda i: (0, 0)),
        ],
        out_specs=(pl.BlockSpec((RB, 272), lambda i: (i, 0)),
                   pl.BlockSpec((16, 8), lambda i: (0, 0))),
        out_shape=(jax.ShapeDtypeStruct((NP, 272), jnp.float32),
                   jax.ShapeDtypeStruct((16, 8), jnp.float32)),
    )(node, wcat, rel, wrelf)


def _mid_body(out1, res1, wcat, x2, y2):
    v = out1[...] + res1[...]
    x = jnp.where(v > 0, v, jnp.exp(v) - 1.0)
    x2[...] = x
    y2[...] = jnp.dot(x, wcat[...], preferred_element_type=jnp.float32,
                      precision=HIGH)


def _mid(out1, res1, wcat):
    return pl.pallas_call(
        _mid_body,
        grid=(4,),
        in_specs=[
            pl.BlockSpec((RB, D), lambda i: (i, 0)),
            pl.BlockSpec((RB, D), lambda i: (i, 0)),
            pl.BlockSpec((D, 144), lambda i: (0, 0)),
        ],
        out_specs=(pl.BlockSpec((RB, D), lambda i: (i, 0)),
                   pl.BlockSpec((RB, 144), lambda i: (i, 0))),
        out_shape=(jax.ShapeDtypeStruct((NP, D), jnp.float32),
                   jax.ShapeDtypeStruct((NP, 144), jnp.float32)),
    )(out1, res1, wcat)


def _fin_body(out2, x2, lng, lnb, wout, y):
    v = out2[...] + x2[...]
    x = jnp.where(v > 0, v, jnp.exp(v) - 1.0)
    mu = jnp.mean(x, axis=1, keepdims=True)
    xc = x - mu
    var = jnp.mean(xc * xc, axis=1, keepdims=True)
    xn = xc * lax.rsqrt(var + 1e-5) * lng[...][None, :] + lnb[...][None, :]
    y[...] = jnp.dot(xn, wout[...], preferred_element_type=jnp.float32,
                     precision=HIGH)


def _fin(out2, x2, lng, lnb, wout8):
    return pl.pallas_call(
        _fin_body,
        grid=(4,),
        in_specs=[
            pl.BlockSpec((RB, D), lambda i: (i, 0)),
            pl.BlockSpec((RB, D), lambda i: (i, 0)),
            pl.BlockSpec((D,), lambda i: (0,)),
            pl.BlockSpec((D,), lambda i: (0,)),
            pl.BlockSpec((D, 8), lambda i: (0, 0)),
        ],
        out_specs=pl.BlockSpec((RB, 8), lambda i: (i, 0)),
        out_shape=jax.ShapeDtypeStruct((NP, 8), jnp.float32),
    )(out2, x2, lng, lnb, wout8)


def _gdt_layer_kernel(has_rel):
    nsc = 10 if has_rel else 8

    def body(*refs):
        if has_rel:
            (s_tab, t_tab, rel_hm, src_h, dst_h, et_h, offs_h, f0_h,
             attn_h, fa_h, fo_h,
             s_loc, t_loc, r_loc, den, out2d, rows2d,
             src_v, dst_v, et_v, ex_v, attn_v, idx_v, offs_v, sem) = refs
        else:
            (s_tab, t_tab, src_h, dst_h, offs_h, f0_h,
             attn_h, fa_h, fo_h,
             s_loc, t_loc, den, out2d, rows2d,
             src_v, dst_v, ex_v, attn_v, idx_v, offs_v, sem) = refs
            rel_hm = et_h = r_loc = et_v = None

        cid = lax.axis_index("c")
        sid = lax.axis_index("s")
        h = cid * 4 + lax.rem(sid, 4)
        r = lax.div(sid, 4)
        nbase = r * RSPAN

        pltpu.sync_copy(s_tab.at[h], s_loc)
        pltpu.sync_copy(t_tab.at[h], t_loc)
        if has_rel:
            pltpu.sync_copy(rel_hm.at[h], r_loc)
        pltpu.sync_copy(offs_h, offs_v)
        rowbase = pl.multiple_of(h * NP + nbase, 8)

        lanes = lax.iota(jnp.int32, 16)
        ov = offs_v[...]
        start = jnp.sum(jnp.where(lanes == r, ov, 0))
        count = jnp.sum(jnp.where(lanes == r + 4, ov, 0))
        n16 = jnp.sum(jnp.where(lanes == 8, ov, 0))
        nrs = jnp.sum(jnp.where(lanes == 9, ov, 0))
        ng = jnp.sum(jnp.where(lanes == 10, ov, 0))
        nhop = jnp.sum(jnp.where(lanes == 11, ov, 0))
        nrs2 = jnp.sum(jnp.where(lanes == 12, ov, 0))
        nblk = lax.shift_right_logical(count + (BLK - 1), 11)

        def zden(i, _):
            den[pl.ds(i * 16, 16)] = jnp.zeros((16,), jnp.float32)
            return 0
        lax.fori_loop(0, n16, zden, 0)

        def p1(b, _):
            ebase = pl.multiple_of(start + b * BLK, 8)
            pltpu.sync_copy(src_h.at[pl.ds(ebase, BLK)], src_v)
            pltpu.sync_copy(dst_h.at[pl.ds(ebase, BLK)], dst_v)
            if has_rel:
                pltpu.sync_copy(et_h.at[pl.ds(ebase, BLK)], et_v)
            def p1g(i, _):
                sl = pl.ds(pl.multiple_of(i * 16, 16), 16)
                sv = src_v[sl]
                dv = dst_v[sl]
                z = plsc.load_gather(s_loc, [sv]) + plsc.load_gather(t_loc, [dv])
                if has_rel:
                    z = z + plsc.load_gather(r_loc, [et_v[sl]])
                zl = jnp.where(z >= 0, z, 0.2 * z)
                ex = jnp.exp(zl)
                ex_v[sl] = ex
                msk = (b * BLK + i * 16 + lanes) < count
                plsc.addupdate_scatter(den, [dv - nbase], ex, mask=msk)
                return 0
            lax.fori_loop(0, ng, p1g, 0)
            pltpu.sync_copy(ex_v, attn_h.at[h, pl.ds(ebase, BLK)])
            return 0
        lax.fori_loop(0, nblk, p1, 0)

        def dinv(i, _):
            sl = pl.ds(i * 16, 16)
            den[sl] = 1.0 / (den[sl] + 1e-16)
            return 0
        lax.fori_loop(0, n16, dinv, 0)

        def p2(b, _):
            ebase = pl.multiple_of(start + b * BLK, 8)
            pltpu.sync_copy(dst_h.at[pl.ds(ebase, BLK)], dst_v)
            pltpu.sync_copy(attn_h.at[h, pl.ds(ebase, BLK)], ex_v)
            def p2g(i, _):
                sl = pl.ds(pl.multiple_of(i * 16, 16), 16)
                dl = dst_v[sl] - nbase
                msk = (b * BLK + i * 16 + lanes) < count
                dv = plsc.load_gather(den, [dl], mask=msk)
                attn_v[sl] = ex_v[sl] * dv
                return 0
            lax.fori_loop(0, ng, p2g, 0)
            pltpu.sync_copy(attn_v, attn_h.at[h, pl.ds(ebase, BLK)])
            return 0
        lax.fori_loop(0, nblk, p2, 0)

        for c in range(2):
            cb = c * (RSPAN // 2)
            pltpu.sync_copy(f0_h.at[pl.ds(rowbase + cb, RSPAN // 2), :],
                            rows2d.at[pl.ds(0, RSPAN // 2), :])
            pltpu.sync_copy(rows2d.at[pl.ds(0, RSPAN // 2), :],
                            fa_h.at[pl.ds(rowbase + cb, RSPAN // 2), :])
        plsc.subcore_barrier()

        def hoploop(k, _):
            def zout(i, _):
                out2d[i] = jnp.zeros((16,), jnp.float32)
                return 0
            lax.fori_loop(0, nrs, zout, 0)

            def hop(b, _):
                ebase = pl.multiple_of(start + b * BLK, 8)
                pltpu.sync_copy(src_h.at[pl.ds(ebase, BLK)], src_v)
                pltpu.sync_copy(dst_h.at[pl.ds(ebase, BLK)], dst_v)
                pltpu.sync_copy(attn_h.at[h, pl.ds(ebase, BLK)], attn_v)
                hoff = h * NP
                def idxg(i, _):
                    sl = pl.ds(pl.multiple_of(i * 16, 16), 16)
                    idx_v[sl] = src_v[sl] + hoff
                    return 0
                lax.fori_loop(0, ng, idxg, 0)
                cps = [pltpu.async_copy(
                    fa_h.at[idx_v.at[pl.ds(kk * 128, 128)]],
                    rows2d.at[pl.ds(kk * 128, 128), :], sem)
                    for kk in range(BLK // 128)]
                for c in cps:
                    c.wait()
                def hopg(i, _):
                    sl = pl.ds(pl.multiple_of(i * 16, 16), 16)
                    dl = dst_v[sl] - nbase
                    av = attn_v[sl]
                    msk = (b * BLK + i * 16 + lanes) < count
                    rbase = i * 16
                    for j in range(16):
                        fv = plsc.load_gather(
                            rows2d, [rbase + lanes,
                                     jnp.full((16,), j, jnp.int32)])
                        plsc.addupdate_scatter(
                            out2d, [dl, jnp.full((16,), j, jnp.int32)],
                            fv * av, mask=msk)
                    return 0
                lax.fori_loop(0, ng, hopg, 0)
                return 0
            lax.fori_loop(0, nblk, hop, 0)

            for c in range(2):
                cb = c * (RSPAN // 2)
                pltpu.sync_copy(f0_h.at[pl.ds(rowbase + cb, RSPAN // 2), :],
                                rows2d.at[pl.ds(0, RSPAN // 2), :])
                def comb(i, _):
                    out2d[cb + i] = ALPHA * rows2d[i] + A1M * out2d[cb + i]
                    return 0
                lax.fori_loop(0, nrs2, comb, 0)
            plsc.subcore_barrier()
            pltpu.sync_copy(out2d, fa_h.at[pl.ds(rowbase, RSPAN), :])
            plsc.subcore_barrier()
            return 0

        lax.fori_loop(0, nhop, hoploop, 0)
        pltpu.sync_copy(out2d, fo_h.at[pl.ds(rowbase, RSPAN), :])

    scr = [
        pltpu.VMEM((NP,), jnp.float32),
        pltpu.VMEM((NP,), jnp.float32),
    ]
    if has_rel:
        scr.append(pltpu.VMEM((16,), jnp.float32))
    scr += [
        pltpu.VMEM((RSPAN,), jnp.float32),
        pltpu.VMEM((RSPAN, DH), jnp.float32),
        pltpu.VMEM((BLK, DH), jnp.float32),
        pltpu.VMEM((BLK,), jnp.int32),
        pltpu.VMEM((BLK,), jnp.int32),
    ]
    if has_rel:
        scr.append(pltpu.VMEM((BLK,), jnp.int32))
    scr += [
        pltpu.VMEM((BLK,), jnp.float32),
        pltpu.VMEM((BLK,), jnp.float32),
        pltpu.VMEM((BLK,), jnp.int32),
        pltpu.VMEM((16,), jnp.int32),
        pltpu.SemaphoreType.DMA,
    ]
    out_type = (
        jax.ShapeDtypeStruct((H, ECAP), jnp.float32),
        jax.ShapeDtypeStruct((H * NP, DH), jnp.float32),
        jax.ShapeDtypeStruct((H * NP, DH), jnp.float32),
    )
    return pl.kernel(body, mesh=_MESH, compiler_params=_SC_PARAMS,
                     out_type=out_type, scratch_types=scr)


_gdt_l1 = _gdt_layer_kernel(True)
_gdt_l2 = _gdt_layer_kernel(False)


@functools.partial(
    pl.kernel, mesh=_MESH, compiler_params=_SC_PARAMS,
    out_type=jax.ShapeDtypeStruct((NCAND,), jnp.float32),
    scratch_types=[
        pltpu.VMEM((NP,), jnp.float32),
        pltpu.VMEM((64,), jnp.int32),
        pltpu.VMEM((64,), jnp.int32),
        pltpu.VMEM((64,), jnp.float32),
        pltpu.VMEM((16,), jnp.float32),
    ],
)
def _cand_kernel(nw_hbm, cs_hbm, ce_hbm, bout_hbm, out_hbm,
                 nw_loc, cs_v, ce_v, o_v, b_v):
    wid = lax.axis_index("s") * 2 + lax.axis_index("c")
    base = pl.multiple_of(wid * 64, 8)
    pltpu.sync_copy(nw_hbm, nw_loc)
    pltpu.sync_copy(cs_hbm.at[pl.ds(base, 64)], cs_v)
    pltpu.sync_copy(ce_hbm.at[pl.ds(base, 64)], ce_v)
    pltpu.sync_copy(bout_hbm, b_v)
    bv = b_v[...]
    for i in range(4):
        sl = pl.ds(i * 16, 16)
        sv = plsc.load_gather(nw_loc, [cs_v[sl] + 1])
        ev = plsc.load_gather(nw_loc, [ce_v[sl] - 1])
        o_v[sl] = 0.5 * (sv + ev) + bv
    pltpu.sync_copy(o_v, out_hbm.at[pl.ds(base, 64)])


def _fold(w, a):
    return (w.reshape(-1, H, DH) * a[None]).sum(-1)


def kernel(seq_inputs, seq_lens, seq_mask, edge_index, edge_type, cand_start,
           cand_end, emb_table, pos_table, W_f, U_f, b_f, W_b, U_b, b_b,
           rel_embed, W_src1, W_dst1, W_rel1, a1, W_res1, W_src2, W_dst2, a2,
           ln_g, ln_b, W_out, b_out):
    f32 = jnp.float32
    src = edge_index[0].astype(jnp.int32)
    dst = edge_index[1].astype(jnp.int32)
    et = edge_type.astype(jnp.int32)

    r_e = dst // RSPAN
    onehot = (r_e[:, None] == jnp.arange(4)[None, :]).astype(jnp.int32)
    counts = onehot.sum(0)
    rank = jnp.take_along_axis(jnp.cumsum(onehot, axis=0),
                               r_e[:, None], axis=1)[:, 0] - 1
    cap = ((counts + BLK - 1) // BLK) * BLK
    offs_p = jnp.concatenate([jnp.zeros((1,), jnp.int32),
                              jnp.cumsum(cap)]).astype(jnp.int32)
    pos_e = offs_p[r_e] + rank
    slot = jnp.arange(ECAP, dtype=jnp.int32)
    r_slot = (slot[:, None] >= offs_p[None, 1:4]).astype(jnp.int32).sum(1)
    packed = jnp.stack([src, dst, et], axis=1)
    dflt = jnp.stack([jnp.zeros((ECAP,), jnp.int32), r_slot * RSPAN,
                      jnp.zeros((ECAP,), jnp.int32)], axis=1)
    packed_s = dflt.at[pos_e].set(packed)
    src_s, dst_s, et_s = packed_s[:, 0], packed_s[:, 1], packed_s[:, 2]
    offs16 = jnp.zeros((16,), jnp.int32).at[:4].set(offs_p[:4]).at[4:8].set(counts).at[8:13].set(
        jnp.array([RSPAN // 16, RSPAN, BLK // 16, HOP, RSPAN // 2], jnp.int32))

    idx_tm = jnp.pad(seq_inputs.astype(jnp.int32).T, ((0, 0), (0, BPAD - B)))
    idx_tm = jnp.pad(idx_tm.reshape(-1), (0, 288))
    xg = _emb_kernel(emb_table.astype(f32), idx_tm)[:L * BPAD].reshape(
        L, BPAD, D)
    wuf = jnp.concatenate([W_f, U_f], axis=0).astype(f32)
    wub = jnp.concatenate([W_b, U_b], axis=0).astype(f32)
    hcat = _lstm(xg, pos_table.astype(f32), wuf, b_f.astype(f32), wub,
                 b_b.astype(f32))
    node = hcat.transpose(1, 0, 2)[:B].reshape(N, 2 * D)
    node = jnp.pad(node, ((0, NP - N), (0, 0)))

    ws1 = _fold(W_src1, a1)
    wt1 = _fold(W_dst1, a1)
    wrelf = _fold(W_rel1, a1)
    wcat1 = jnp.concatenate([ws1, wt1, W_dst1, W_res1], axis=1).astype(f32)
    y1, r1 = _proj1(node, wcat1, rel_embed.astype(f32), wrelf.astype(f32))
    s1t = y1[:, 0:8].T
    t1t = y1[:, 8:16].T
    f01 = y1[:, 16:144].reshape(NP, H, DH).transpose(1, 0, 2).reshape(H * NP, DH)
    res1 = y1[:, 144:272]
    rel_hm = r1.T

    _, _, feat1 = _gdt_l1(s1t, t1t, rel_hm, src_s, dst_s, et_s, offs16, f01)
    out1 = feat1.reshape(H, NP, DH).transpose(1, 0, 2).reshape(NP, H * DH)

    ws2 = _fold(W_src2, a2)
    wt2 = _fold(W_dst2, a2)
    wcat2 = jnp.concatenate([ws2, wt2, W_dst2], axis=1).astype(f32)
    x2, y2 = _mid(out1, res1, wcat2)
    s2t = y2[:, 0:8].T
    t2t = y2[:, 8:16].T
    f02 = y2[:, 16:144].reshape(NP, H, DH).transpose(1, 0, 2).reshape(H * NP, DH)

    _, _, feat2 = _gdt_l2(s2t, t2t, src_s, dst_s, offs16, f02)
    out2 = feat2.reshape(H, NP, DH).transpose(1, 0, 2).reshape(NP, H * DH)

    wout8 = jnp.pad(W_out.astype(f32), ((0, 0), (0, 7)))
    y3 = _fin(out2, x2, ln_g.astype(f32), ln_b.astype(f32), wout8)
    nodew = y3[:, 0]
    bvec = jnp.broadcast_to(b_out.astype(f32), (16,))
    cand = _cand_kernel(nodew, cand_start.astype(jnp.int32),
                        cand_end.astype(jnp.int32), bvec)
    return cand[:, None]


# speedup vs baseline: 26.9293x; 1.3966x over previous
import functools

import jax
import jax.numpy as jnp
from jax import lax
from jax.experimental import pallas as pl
from jax.experimental.pallas import tpu as pltpu
from jax.experimental.pallas import tpu_sc as plsc

B, L = 20, 500
D = 128
H, DH = 8, 16
E = 320000
N = B * L
NP = 10240
RSPAN = 2560
BLK = 2048
ECAP = E + 4 * BLK
NCAND = 2048
HOP = 4
ALPHA = 0.15
A1M = 1.0 - ALPHA
BPAD = 24
HIGH = lax.Precision.HIGHEST

_SC_PARAMS = pltpu.CompilerParams(needs_layout_passes=False,
                                  use_tc_tiling_on_sc=False)
_MESH = plsc.VectorSubcoreMesh(core_axis_name="c", subcore_axis_name="s")


@functools.partial(
    pl.kernel, mesh=_MESH, compiler_params=_SC_PARAMS,
    out_type=jax.ShapeDtypeStruct((12288, D), jnp.float32),
    scratch_types=[
        pltpu.VMEM((384,), jnp.int32),
        pltpu.VMEM((384, D), jnp.float32),
        pltpu.SemaphoreType.DMA,
    ],
)
def _emb_kernel(emb_hbm, idx_hbm, out_hbm, idx_v, rows_v, sem):
    wid = lax.axis_index("s") * 2 + lax.axis_index("c")
    base = pl.multiple_of(wid * 384, 8)
    pltpu.sync_copy(idx_hbm.at[pl.ds(base, 384)], idx_v)
    cps = [pltpu.async_copy(emb_hbm.at[idx_v.at[pl.ds(k * 128, 128)]],
                            rows_v.at[pl.ds(k * 128, 128), :], sem)
           for k in range(3)]
    for c in cps:
        c.wait()
    pltpu.sync_copy(rows_v, out_hbm.at[pl.ds(base, 384), :])


def _lstm_body(xg, pos, wuf, bf, wub, bb, out, hf, cf, hb, cb):
    z24 = jnp.zeros((BPAD, D), jnp.float32)
    hf[...] = z24
    cf[...] = z24
    hb[...] = z24
    cb[...] = z24

    def gates(cat, wu, b):
        z = jnp.dot(cat, wu[...], preferred_element_type=jnp.float32,
                    precision=HIGH) + b[...][None, :]
        zi = z[:, 0:D]
        zf = z[:, D:2 * D]
        zg = z[:, 2 * D:3 * D]
        zo = z[:, 3 * D:4 * D]
        i = jax.nn.sigmoid(zi)
        f = jax.nn.sigmoid(zf)
        g = jnp.tanh(zg)
        o = jax.nn.sigmoid(zo)
        return i, f, g, o

    def step(t, _):
        tb = (L - 1) - t
        xt = xg[t] + pos[t]
        xb = xg[tb] + pos[tb]
        catf = jnp.concatenate([xt, hf[...]], axis=1)
        catb = jnp.concatenate([xb, hb[...]], axis=1)
        fi, ff, fg, fo = gates(catf, wuf, bf)
        bi, bf_, bg, bo = gates(catb, wub, bb)
        cfn = ff * cf[...] + fi * fg
        hfn = fo * jnp.tanh(cfn)
        cbn = bf_ * cb[...] + bi * bg
        hbn = bo * jnp.tanh(cbn)
        cf[...] = cfn
        hf[...] = hfn
        cb[...] = cbn
        hb[...] = hbn
        out[t, :, 0:D] = hfn
        out[tb, :, D:2 * D] = hbn
        return 0

    lax.fori_loop(0, L, step, 0)


def _lstm(xg, pos, wuf, bf, wub, bb):
    return pl.pallas_call(
        _lstm_body,
        out_shape=jax.ShapeDtypeStruct((L, BPAD, 2 * D), jnp.float32),
        scratch_shapes=[pltpu.VMEM((BPAD, D), jnp.float32) for _ in range(4)],
    )(xg, pos, wuf, bf, wub, bb)


RB = NP // 4


def _proj1_body(node, wcat, rel, wrelf, y, r1):
    y[...] = jnp.dot(node[...], wcat[...],
                     preferred_element_type=jnp.float32, precision=HIGH)

    @pl.when(pl.program_id(0) == 0)
    def _():
        r1[...] = jnp.dot(rel[...], wrelf[...],
                          preferred_element_type=jnp.float32, precision=HIGH)


def _proj1(node, wcat, rel, wrelf):
    return pl.pallas_call(
        _proj1_body,
        grid=(4,),
        in_specs=[
            pl.BlockSpec((RB, 2 * D), lambda i: (i, 0)),
            pl.BlockSpec((2 * D, 272), lambda i: (0, 0)),
            pl.BlockSpec((16, 64), lambda i: (0, 0)),
            pl.BlockSpec((64, 8), lambda i: (0, 0)),
        ],
        out_specs=(pl.BlockSpec((RB, 272), lambda i: (i, 0)),
                   pl.BlockSpec((16, 8), lambda i: (0, 0))),
        out_shape=(jax.ShapeDtypeStruct((NP, 272), jnp.float32),
                   jax.ShapeDtypeStruct((16, 8), jnp.float32)),
    )(node, wcat, rel, wrelf)


def _mid_body(out1, res1, wcat, x2, y2):
    v = out1[...] + res1[...]
    x = jnp.where(v > 0, v, jnp.exp(v) - 1.0)
    x2[...] = x
    y2[...] = jnp.dot(x, wcat[...], preferred_element_type=jnp.float32,
                      precision=HIGH)


def _mid(out1, res1, wcat):
    return pl.pallas_call(
        _mid_body,
        grid=(4,),
        in_specs=[
            pl.BlockSpec((RB, D), lambda i: (i, 0)),
            pl.BlockSpec((RB, D), lambda i: (i, 0)),
            pl.BlockSpec((D, 144), lambda i: (0, 0)),
        ],
        out_specs=(pl.BlockSpec((RB, D), lambda i: (i, 0)),
                   pl.BlockSpec((RB, 144), lambda i: (i, 0))),
        out_shape=(jax.ShapeDtypeStruct((NP, D), jnp.float32),
                   jax.ShapeDtypeStruct((NP, 144), jnp.float32)),
    )(out1, res1, wcat)


def _fin_body(out2, x2, lng, lnb, wout, y):
    v = out2[...] + x2[...]
    x = jnp.where(v > 0, v, jnp.exp(v) - 1.0)
    mu = jnp.mean(x, axis=1, keepdims=True)
    xc = x - mu
    var = jnp.mean(xc * xc, axis=1, keepdims=True)
    xn = xc * lax.rsqrt(var + 1e-5) * lng[...][None, :] + lnb[...][None, :]
    y[...] = jnp.dot(xn, wout[...], preferred_element_type=jnp.float32,
                     precision=HIGH)


def _fin(out2, x2, lng, lnb, wout8):
    return pl.pallas_call(
        _fin_body,
        grid=(4,),
        in_specs=[
            pl.BlockSpec((RB, D), lambda i: (i, 0)),
            pl.BlockSpec((RB, D), lambda i: (i, 0)),
            pl.BlockSpec((D,), lambda i: (0,)),
            pl.BlockSpec((D,), lambda i: (0,)),
            pl.BlockSpec((D, 8), lambda i: (0, 0)),
        ],
        out_specs=pl.BlockSpec((RB, 8), lambda i: (i, 0)),
        out_shape=jax.ShapeDtypeStruct((NP, 8), jnp.float32),
    )(out2, x2, lng, lnb, wout8)


def _gdt_layer_kernel(has_rel):
    nsc = 10 if has_rel else 8

    def body(*refs):
        if has_rel:
            (s_tab, t_tab, rel_hm, src_h, dst_h, et_h, offs_h, f0_h,
             attn_h, fa_h, fo_h,
             s_loc, t_loc, r_loc, den, out2d, rows2d,
             src_v, dst_v, et_v, ex_v, attn_v, idx_v, offs_v, sem) = refs
        else:
            (s_tab, t_tab, src_h, dst_h, offs_h, f0_h,
             attn_h, fa_h, fo_h,
             s_loc, t_loc, den, out2d, rows2d,
             src_v, dst_v, ex_v, attn_v, idx_v, offs_v, sem) = refs
            rel_hm = et_h = r_loc = et_v = None

        cid = lax.axis_index("c")
        sid = lax.axis_index("s")
        h = cid * 4 + lax.rem(sid, 4)
        r = lax.div(sid, 4)
        nbase = r * RSPAN

        pltpu.sync_copy(s_tab.at[h], s_loc)
        pltpu.sync_copy(t_tab.at[h], t_loc)
        if has_rel:
            pltpu.sync_copy(rel_hm.at[h], r_loc)
        pltpu.sync_copy(offs_h, offs_v)
        rowbase = pl.multiple_of(h * NP + nbase, 8)

        lanes = lax.iota(jnp.int32, 16)
        ov = offs_v[...]
        start = jnp.sum(jnp.where(lanes == r, ov, 0))
        count = jnp.sum(jnp.where(lanes == r + 4, ov, 0))
        n16 = jnp.sum(jnp.where(lanes == 8, ov, 0))
        nrs = jnp.sum(jnp.where(lanes == 9, ov, 0))
        ng = jnp.sum(jnp.where(lanes == 10, ov, 0))
        nhop = jnp.sum(jnp.where(lanes == 11, ov, 0))
        nrs2 = jnp.sum(jnp.where(lanes == 12, ov, 0))
        nblk = lax.shift_right_logical(count + (BLK - 1), 11)

        @plsc.parallel_loop(0, n16, 1, unroll=8)
        def zden(i):
            den[pl.ds(pl.multiple_of(i * 16, 16), 16)] = jnp.zeros(
                (16,), jnp.float32)

        def p1(b, _):
            ebase = pl.multiple_of(start + b * BLK, 8)
            pltpu.sync_copy(src_h.at[pl.ds(ebase, BLK)], src_v)
            pltpu.sync_copy(dst_h.at[pl.ds(ebase, BLK)], dst_v)
            if has_rel:
                pltpu.sync_copy(et_h.at[pl.ds(ebase, BLK)], et_v)
            @plsc.parallel_loop(0, ng, 1, unroll=4)
            def p1g(i):
                sl = pl.ds(pl.multiple_of(i * 16, 16), 16)
                sv = src_v[sl]
                dv = dst_v[sl]
                z = plsc.load_gather(s_loc, [sv]) + plsc.load_gather(t_loc, [dv])
                if has_rel:
                    z = z + plsc.load_gather(r_loc, [et_v[sl]])
                zl = jnp.where(z >= 0, z, 0.2 * z)
                ex = jnp.exp(zl)
                ex_v[sl] = ex
                msk = (b * BLK + i * 16 + lanes) < count
                plsc.addupdate_scatter(den, [dv - nbase], ex, mask=msk)
            pltpu.sync_copy(ex_v, attn_h.at[h, pl.ds(ebase, BLK)])
            return 0
        lax.fori_loop(0, nblk, p1, 0)

        @plsc.parallel_loop(0, n16, 1, unroll=8)
        def dinv(i):
            sl = pl.ds(pl.multiple_of(i * 16, 16), 16)
            den[sl] = 1.0 / (den[sl] + 1e-16)

        def p2(b, _):
            ebase = pl.multiple_of(start + b * BLK, 8)
            pltpu.sync_copy(dst_h.at[pl.ds(ebase, BLK)], dst_v)
            pltpu.sync_copy(attn_h.at[h, pl.ds(ebase, BLK)], ex_v)
            @plsc.parallel_loop(0, ng, 1, unroll=4)
            def p2g(i):
                sl = pl.ds(pl.multiple_of(i * 16, 16), 16)
                dl = dst_v[sl] - nbase
                msk = (b * BLK + i * 16 + lanes) < count
                dv = plsc.load_gather(den, [dl], mask=msk)
                attn_v[sl] = ex_v[sl] * dv
            pltpu.sync_copy(attn_v, attn_h.at[h, pl.ds(ebase, BLK)])
            return 0
        lax.fori_loop(0, nblk, p2, 0)

        for c in range(2):
            cb = c * (RSPAN // 2)
            pltpu.sync_copy(f0_h.at[pl.ds(rowbase + cb, RSPAN // 2), :],
                            rows2d.at[pl.ds(0, RSPAN // 2), :])
            pltpu.sync_copy(rows2d.at[pl.ds(0, RSPAN // 2), :],
                            fa_h.at[pl.ds(rowbase + cb, RSPAN // 2), :])
        plsc.subcore_barrier()

        def hoploop(k, _):
            @plsc.parallel_loop(0, nrs, 1, unroll=8)
            def zout(i):
                out2d[i] = jnp.zeros((16,), jnp.float32)

            def hop(b, _):
                ebase = pl.multiple_of(start + b * BLK, 8)
                pltpu.sync_copy(src_h.at[pl.ds(ebase, BLK)], src_v)
                pltpu.sync_copy(dst_h.at[pl.ds(ebase, BLK)], dst_v)
                pltpu.sync_copy(attn_h.at[h, pl.ds(ebase, BLK)], attn_v)
                hoff = h * NP
                @plsc.parallel_loop(0, ng, 1, unroll=8)
                def idxg(i):
                    sl = pl.ds(pl.multiple_of(i * 16, 16), 16)
                    idx_v[sl] = src_v[sl] + hoff
                cps = [pltpu.async_copy(
                    fa_h.at[idx_v.at[pl.ds(kk * 128, 128)]],
                    rows2d.at[pl.ds(kk * 128, 128), :], sem)
                    for kk in range(BLK // 128)]
                for c in cps:
                    c.wait()
                @plsc.parallel_loop(0, ng, 1, unroll=2)
                def hopg(i):
                    sl = pl.ds(pl.multiple_of(i * 16, 16), 16)
                    dl = dst_v[sl] - nbase
                    av = attn_v[sl]
                    msk = (b * BLK + i * 16 + lanes) < count
                    rbase = i * 16
                    for j in range(16):
                        fv = plsc.load_gather(
                            rows2d, [rbase + lanes,
                                     jnp.full((16,), j, jnp.int32)])
                        plsc.addupdate_scatter(
                            out2d, [dl, jnp.full((16,), j, jnp.int32)],
                            fv * av, mask=msk)
                return 0
            lax.fori_loop(0, nblk, hop, 0)

            for c in range(2):
                cb = c * (RSPAN // 2)
                pltpu.sync_copy(f0_h.at[pl.ds(rowbase + cb, RSPAN // 2), :],
                                rows2d.at[pl.ds(0, RSPAN // 2), :])
                @plsc.parallel_loop(0, nrs2, 1, unroll=8)
                def comb(i):
                    out2d[cb + i] = ALPHA * rows2d[i] + A1M * out2d[cb + i]
            plsc.subcore_barrier()
            pltpu.sync_copy(out2d, fa_h.at[pl.ds(rowbase, RSPAN), :])
            plsc.subcore_barrier()
            return 0

        lax.fori_loop(0, nhop, hoploop, 0)
        pltpu.sync_copy(out2d, fo_h.at[pl.ds(rowbase, RSPAN), :])

    scr = [
        pltpu.VMEM((NP,), jnp.float32),
        pltpu.VMEM((NP,), jnp.float32),
    ]
    if has_rel:
        scr.append(pltpu.VMEM((16,), jnp.float32))
    scr += [
        pltpu.VMEM((RSPAN,), jnp.float32),
        pltpu.VMEM((RSPAN, DH), jnp.float32),
        pltpu.VMEM((BLK, DH), jnp.float32),
        pltpu.VMEM((BLK,), jnp.int32),
        pltpu.VMEM((BLK,), jnp.int32),
    ]
    if has_rel:
        scr.append(pltpu.VMEM((BLK,), jnp.int32))
    scr += [
        pltpu.VMEM((BLK,), jnp.float32),
        pltpu.VMEM((BLK,), jnp.float32),
        pltpu.VMEM((BLK,), jnp.int32),
        pltpu.VMEM((16,), jnp.int32),
        pltpu.SemaphoreType.DMA,
    ]
    out_type = (
        jax.ShapeDtypeStruct((H, ECAP), jnp.float32),
        jax.ShapeDtypeStruct((H * NP, DH), jnp.float32),
        jax.ShapeDtypeStruct((H * NP, DH), jnp.float32),
    )
    return pl.kernel(body, mesh=_MESH, compiler_params=_SC_PARAMS,
                     out_type=out_type, scratch_types=scr)


_gdt_l1 = _gdt_layer_kernel(True)
_gdt_l2 = _gdt_layer_kernel(False)


@functools.partial(
    pl.kernel, mesh=_MESH, compiler_params=_SC_PARAMS,
    out_type=jax.ShapeDtypeStruct((NCAND,), jnp.float32),
    scratch_types=[
        pltpu.VMEM((NP,), jnp.float32),
        pltpu.VMEM((64,), jnp.int32),
        pltpu.VMEM((64,), jnp.int32),
        pltpu.VMEM((64,), jnp.float32),
        pltpu.VMEM((16,), jnp.float32),
    ],
)
def _cand_kernel(nw_hbm, cs_hbm, ce_hbm, bout_hbm, out_hbm,
                 nw_loc, cs_v, ce_v, o_v, b_v):
    wid = lax.axis_index("s") * 2 + lax.axis_index("c")
    base = pl.multiple_of(wid * 64, 8)
    pltpu.sync_copy(nw_hbm, nw_loc)
    pltpu.sync_copy(cs_hbm.at[pl.ds(base, 64)], cs_v)
    pltpu.sync_copy(ce_hbm.at[pl.ds(base, 64)], ce_v)
    pltpu.sync_copy(bout_hbm, b_v)
    bv = b_v[...]
    for i in range(4):
        sl = pl.ds(i * 16, 16)
        sv = plsc.load_gather(nw_loc, [cs_v[sl] + 1])
        ev = plsc.load_gather(nw_loc, [ce_v[sl] - 1])
        o_v[sl] = 0.5 * (sv + ev) + bv
    pltpu.sync_copy(o_v, out_hbm.at[pl.ds(base, 64)])


def _fold(w, a):
    return (w.reshape(-1, H, DH) * a[None]).sum(-1)


def kernel(seq_inputs, seq_lens, seq_mask, edge_index, edge_type, cand_start,
           cand_end, emb_table, pos_table, W_f, U_f, b_f, W_b, U_b, b_b,
           rel_embed, W_src1, W_dst1, W_rel1, a1, W_res1, W_src2, W_dst2, a2,
           ln_g, ln_b, W_out, b_out):
    f32 = jnp.float32
    src = edge_index[0].astype(jnp.int32)
    dst = edge_index[1].astype(jnp.int32)
    et = edge_type.astype(jnp.int32)

    r_e = dst // RSPAN
    onehot = (r_e[:, None] == jnp.arange(4)[None, :]).astype(jnp.int32)
    counts = onehot.sum(0)
    rank = jnp.take_along_axis(jnp.cumsum(onehot, axis=0),
                               r_e[:, None], axis=1)[:, 0] - 1
    cap = ((counts + BLK - 1) // BLK) * BLK
    offs_p = jnp.concatenate([jnp.zeros((1,), jnp.int32),
                              jnp.cumsum(cap)]).astype(jnp.int32)
    pos_e = offs_p[r_e] + rank
    slot = jnp.arange(ECAP, dtype=jnp.int32)
    r_slot = (slot[:, None] >= offs_p[None, 1:4]).astype(jnp.int32).sum(1)
    packed = jnp.stack([src, dst, et], axis=1)
    dflt = jnp.stack([jnp.zeros((ECAP,), jnp.int32), r_slot * RSPAN,
                      jnp.zeros((ECAP,), jnp.int32)], axis=1)
    packed_s = dflt.at[pos_e].set(packed)
    src_s, dst_s, et_s = packed_s[:, 0], packed_s[:, 1], packed_s[:, 2]
    offs16 = jnp.zeros((16,), jnp.int32).at[:4].set(offs_p[:4]).at[4:8].set(counts).at[8:13].set(
        jnp.array([RSPAN // 16, RSPAN, BLK // 16, HOP, RSPAN // 2], jnp.int32))

    idx_tm = jnp.pad(seq_inputs.astype(jnp.int32).T, ((0, 0), (0, BPAD - B)))
    idx_tm = jnp.pad(idx_tm.reshape(-1), (0, 288))
    xg = _emb_kernel(emb_table.astype(f32), idx_tm)[:L * BPAD].reshape(
        L, BPAD, D)
    wuf = jnp.concatenate([W_f, U_f], axis=0).astype(f32)
    wub = jnp.concatenate([W_b, U_b], axis=0).astype(f32)
    hcat = _lstm(xg, pos_table.astype(f32), wuf, b_f.astype(f32), wub,
                 b_b.astype(f32))
    node = hcat.transpose(1, 0, 2)[:B].reshape(N, 2 * D)
    node = jnp.pad(node, ((0, NP - N), (0, 0)))

    ws1 = _fold(W_src1, a1)
    wt1 = _fold(W_dst1, a1)
    wrelf = _fold(W_rel1, a1)
    wcat1 = jnp.concatenate([ws1, wt1, W_dst1, W_res1], axis=1).astype(f32)
    y1, r1 = _proj1(node, wcat1, rel_embed.astype(f32), wrelf.astype(f32))
    s1t = y1[:, 0:8].T
    t1t = y1[:, 8:16].T
    f01 = y1[:, 16:144].reshape(NP, H, DH).transpose(1, 0, 2).reshape(H * NP, DH)
    res1 = y1[:, 144:272]
    rel_hm = r1.T

    _, _, feat1 = _gdt_l1(s1t, t1t, rel_hm, src_s, dst_s, et_s, offs16, f01)
    out1 = feat1.reshape(H, NP, DH).transpose(1, 0, 2).reshape(NP, H * DH)

    ws2 = _fold(W_src2, a2)
    wt2 = _fold(W_dst2, a2)
    wcat2 = jnp.concatenate([ws2, wt2, W_dst2], axis=1).astype(f32)
    x2, y2 = _mid(out1, res1, wcat2)
    s2t = y2[:, 0:8].T
    t2t = y2[:, 8:16].T
    f02 = y2[:, 16:144].reshape(NP, H, DH).transpose(1, 0, 2).reshape(H * NP, DH)

    _, _, feat2 = _gdt_l2(s2t, t2t, src_s, dst_s, offs16, f02)
    out2 = feat2.reshape(H, NP, DH).transpose(1, 0, 2).reshape(NP, H * DH)

    wout8 = jnp.pad(W_out.astype(f32), ((0, 0), (0, 7)))
    y3 = _fin(out2, x2, ln_g.astype(f32), ln_b.astype(f32), wout8)
    nodew = y3[:, 0]
    bvec = jnp.broadcast_to(b_out.astype(f32), (16,))
    cand = _cand_kernel(nodew, cand_start.astype(jnp.int32),
                        cand_end.astype(jnp.int32), bvec)
    return cand[:, None]
